# baseline (speedup 1.0000x reference)
import functools

import jax
import jax.numpy as jnp
from jax import lax
from jax.experimental import pallas as pl
from jax.experimental.pallas import tpu as pltpu
from jax.experimental.pallas import tpu_sc as plsc

B, N, H = 32, 64, 32
PG = N * N
NDIST, NFEAT = 5, 3
NE = NDIST * NFEAT
TBL_E = 1537
SP_OFF = NDIST * TBL_E
TBL_ROWS = SP_OFF + 512
NC, NS = 2, 16
NW = NC * NS
CPOS = 64
NCHUNK = PG // CPOS
CROWS = CPOS * NE
NSUB = 8
SUB = CROWS // NSUB
HB = B // 2
NCHW = NCHUNK // 2


def _sp_val(s):
    return jnp.clip(jnp.where(s == 0, 1, jnp.where(s > 1, s - 1, s)), 0, NDIST)


def _table_body(e_ref, w_ref, sp_ref, out_ref):
    E = e_ref[...]
    parts = [lax.dot(E, w_ref[d], preferred_element_type=jnp.float32) * (1.0 / 3.0)
             for d in range(NDIST)]
    s = lax.broadcasted_iota(jnp.int32, (512, 1), 0)
    parts.append(sp_ref[...] * _sp_val(s).astype(jnp.float32))
    out_ref[...] = jnp.concatenate(parts, axis=0)


def _sc_body(table, ge, gs, sidx_h, offs_h, acc_out,
             idxe_v, idxs_v, sidx_v, offs_v, rows_v, acc_v, acc_sh,
             gsem, ssem, osem, isem):
    cid = lax.axis_index("c")
    sid = lax.axis_index("s")
    wid = sid * NC + cid
    b_loc = wid // 2
    hf = wid % 2

    def G(g):
        return hf * NCHW + g

    pltpu.sync_copy(sidx_h, sidx_v)
    pltpu.sync_copy(offs_h, offs_v)

    def fire_idx(g):
        p = g % 2
        pltpu.async_copy(ge.at[b_loc, G(g)], idxe_v.at[p], isem)
        pltpu.async_copy(gs.at[b_loc, G(g)], idxs_v.at[p], isem)

    def wait_idx(g):
        p = g % 2
        pltpu.make_async_copy(ge.at[b_loc, G(g)], idxe_v.at[p], isem).wait()
        pltpu.make_async_copy(gs.at[b_loc, G(g)], idxs_v.at[p], isem).wait()

    def compute_idx(g):
        p = g % 2

        def eb(i, carry):
            sl = pl.ds(i * 16, 16)
            idxe_v[p, sl] = idxe_v[p, sl] + offs_v[sl]
            return carry

        lax.fori_loop(0, CROWS // 16, eb, 0)

        def sb(i, carry):
            sl = pl.ds(i * 16, 16)
            idxs_v[p, sl] = idxs_v[p, sl] + jnp.full((16,), SP_OFF, jnp.int32)
            return carry

        lax.fori_loop(0, CPOS // 16, sb, 0)

    def fire_gathers(g):
        p, q = g % 2, g % 3
        for j in range(NSUB):
            pltpu.async_copy(table.at[idxe_v.at[p, pl.ds(j * SUB, SUB)]],
                             rows_v.at[p, pl.ds(j * SUB, SUB)], gsem)
        pltpu.async_copy(table.at[idxs_v.at[p]], acc_v.at[p], gsem)

    def wait_gathers(g):
        p = g % 2
        for j in range(NSUB):
            pltpu.make_async_copy(table.at[idxe_v.at[p, pl.ds(j * SUB, SUB)]],
                                  rows_v.at[p, pl.ds(j * SUB, SUB)], gsem).wait()
        pltpu.make_async_copy(table.at[idxs_v.at[p]], acc_v.at[p], gsem).wait()

    def fire_scatters(g):
        p, q = g % 2, g % 3
        for j in range(NSUB):
            pltpu.async_copy(rows_v.at[p, pl.ds(j * SUB, SUB)],
                             acc_sh.at[sid, q].at[sidx_v.at[j]], ssem, add=True)

    def wait_scatters(g):
        p, q = g % 2, g % 3
        for j in range(NSUB):
            pltpu.make_async_copy(rows_v.at[p, pl.ds(j * SUB, SUB)],
                                  acc_sh.at[sid, q].at[sidx_v.at[j]],
                                  ssem).wait()

    def fire_out(g):
        pltpu.async_copy(acc_sh.at[sid, g % 3],
                         acc_out.at[b_loc, G(g), :, pl.ds(0, H)], osem)

    def wait_out(g):
        pltpu.make_async_copy(acc_sh.at[sid, g % 3],
                              acc_out.at[b_loc, G(g), :, pl.ds(0, H)],
                              osem).wait()

    pltpu.sync_copy(ge.at[b_loc, G(0)], idxe_v.at[0])
    pltpu.sync_copy(gs.at[b_loc, G(0)], idxs_v.at[0])
    compute_idx(0)
    fire_gathers(0)
    fire_idx(1)

    def chunk(g, carry):
        wait_gathers(g)
        pltpu.sync_copy(acc_v.at[g % 2], acc_sh.at[sid, g % 3])
        fire_scatters(g)
        @pl.when(g >= 1)
        def _():
            wait_scatters(g - 1)
            fire_out(g - 1)
        @pl.when(g >= 2)
        def _():
            wait_out(g - 2)
        @pl.when(g <= NCHW - 2)
        def _():
            wait_idx(g + 1)
            compute_idx(g + 1)
            fire_gathers(g + 1)
        @pl.when(g <= NCHW - 3)
        def _():
            fire_idx(g + 2)
        return carry

    lax.fori_loop(0, NCHW, chunk, 0)
    wait_scatters(NCHW - 1)
    fire_out(NCHW - 1)
    wait_out(NCHW - 2)
    wait_out(NCHW - 1)


@functools.cache
def _sc_gather():
    return pl.kernel(
        _sc_body,
        out_type=jax.ShapeDtypeStruct((HB, NCHUNK, CPOS, 4 * H), jnp.float32),
        mesh=plsc.VectorSubcoreMesh(core_axis_name="c", subcore_axis_name="s",
                                    num_cores=NC, num_subcores=NS),
        compiler_params=pltpu.CompilerParams(use_tc_tiling_on_sc=False),
        scratch_types=[
            pltpu.VMEM((2, CROWS), jnp.int32),
            pltpu.VMEM((2, CPOS), jnp.int32),
            pltpu.VMEM((NSUB, SUB), jnp.int32),
            pltpu.VMEM((CROWS,), jnp.int32),
            pltpu.VMEM((2, CROWS, H), jnp.float32),
            pltpu.VMEM((2, CPOS, H), jnp.float32),
            pltpu.VMEM_SHARED((NS, 3, CPOS, H), jnp.float32),
            pltpu.SemaphoreType.DMA,
            pltpu.SemaphoreType.DMA,
            pltpu.SemaphoreType.DMA,
            pltpu.SemaphoreType.DMA,
        ],
    )


def _sc_gather_call(table, ge, gs, sidx, offs):
    return _sc_gather()(table, ge, gs, sidx, offs)


def _final_call(accv, sp_h, ab_h, t_col):
    return pl.pallas_call(
        _final_body,
        grid=(HB,),
        in_specs=[
            pl.BlockSpec((1, N, N, 4 * H), lambda b: (b, 0, 0, 0)),
            pl.BlockSpec((1, N, N), lambda b: (b, 0, 0)),
            pl.BlockSpec((1, N + 1, N + 1), lambda b: (b, 0, 0)),
            pl.BlockSpec((H, 1), lambda b: (0, 0)),
        ],
        out_specs=pl.BlockSpec((1, H, N + 1, N + 1), lambda b: (b, 0, 0, 0)),
        out_shape=jax.ShapeDtypeStruct((HB, H, N + 1, N + 1), jnp.float32),
    )(accv, sp_h, ab_h, t_col)


def _final_body(acc_ref, s_ref, ab_ref, t_ref, out_ref):
    ab2 = 2.0 * ab_ref[0]
    t_col = t_ref[...]
    t_row = jnp.broadcast_to(t_col, (H, N + 1))
    ident = (lax.broadcasted_iota(jnp.int32, (H, H), 0)
             == lax.broadcasted_iota(jnp.int32, (H, H), 1)).astype(jnp.float32)
    lane65 = lax.broadcasted_iota(jnp.int32, (1, N + 1), 1)
    t0mask = jnp.where(lane65 == 0, t_row, 0.0)
    zcol = jnp.zeros((H, 1), jnp.float32)
    out_ref[0, :, 0, :] = ab2[0:1, :] + t_row
    for i in range(N):
        s = s_ref[0, pl.ds(i, 1), :]
        r = jnp.where(s <= 2, 1.0,
                      jnp.where(s == 3, 0.5,
                                jnp.where(s == 4, 1.0 / 3.0,
                                          jnp.where(s == 5, 0.25, 0.2))))
        e = acc_ref[0, i, :, pl.ds(0, H)]
        et = lax.dot_general(ident, e, (((1,), (1,)), ((), ())),
                             preferred_element_type=jnp.float32)
        et = et * r
        res = ab2[i + 1:i + 2, :] + t0mask + jnp.concatenate([zcol, et], axis=1)
        out_ref[0, :, i + 1, :] = res


def kernel(input_nodes, attn_bias, spatial_pos, input_edges, attn_edge_type,
           edge_encoder_w, edge_dis_encoder_w, spatial_pos_encoder_w, gtvd_w):
    w5 = edge_dis_encoder_w.reshape(128, H, H)[:NDIST]

    table = pl.pallas_call(
        _table_body,
        out_shape=jax.ShapeDtypeStruct((TBL_ROWS, H), jnp.float32),
    )(edge_encoder_w, w5, spatial_pos_encoder_w)

    ge = input_edges.reshape(B, NCHUNK, CROWS)
    gs = spatial_pos
    sidx = (jnp.arange(CROWS, dtype=jnp.int32) // NE).reshape(NSUB, SUB)
    offs = ((jnp.arange(CROWS, dtype=jnp.int32) // NFEAT) % NDIST) * TBL_E
    t_col = gtvd_w.reshape(H, 1)

    outs = []
    for h0 in (0, HB):
        accv = _sc_gather_call(table, ge[h0:h0 + HB], gs[h0:h0 + HB],
                               sidx, offs)
        outs.append(_final_call(accv, spatial_pos[h0:h0 + HB],
                                attn_bias[h0:h0 + HB], t_col))
    return jnp.concatenate(outs, axis=0)


# speedup vs baseline: 1.0934x; 1.0934x over previous
import functools

import jax
import jax.numpy as jnp
from jax import lax
from jax.experimental import pallas as pl
from jax.experimental.pallas import tpu as pltpu
from jax.experimental.pallas import tpu_sc as plsc

B, N, H = 32, 64, 32
PG = N * N
NDIST, NFEAT = 5, 3
NE = NDIST * NFEAT
TBL_E = 1537
SP_OFF = NDIST * TBL_E
TBL_ROWS = SP_OFF + 512
NC, NS = 2, 16
NW = NC * NS
CPOS = 64
NCHUNK = PG // CPOS
CROWS = CPOS * NE
NSUB = 8
SUB = CROWS // NSUB


def _sp_val(s):
    return jnp.clip(jnp.where(s == 0, 1, jnp.where(s > 1, s - 1, s)), 0, NDIST)


def _table_body(e_ref, w_ref, sp_ref, out_ref):
    E = e_ref[...]
    parts = [lax.dot(E, w_ref[d], preferred_element_type=jnp.float32) * (1.0 / 3.0)
             for d in range(NDIST)]
    s = lax.broadcasted_iota(jnp.int32, (512, 1), 0)
    parts.append(sp_ref[...] * _sp_val(s).astype(jnp.float32))
    out_ref[...] = jnp.concatenate(parts, axis=0)


def _sc_body(table, ge, gs, sidx_h, offs_h, acc_out,
             idxe_v, idxs_v, sidx_v, offs_v, rows_v, acc_v, acc_sh,
             gsem, ssem, osem, isem):
    cid = lax.axis_index("c")
    sid = lax.axis_index("s")
    wid = sid * NC + cid
    pltpu.sync_copy(sidx_h, sidx_v)
    pltpu.sync_copy(offs_h, offs_v)

    def fire_idx(g):
        p = g % 2
        pltpu.async_copy(ge.at[wid, g], idxe_v.at[p], isem)
        pltpu.async_copy(gs.at[wid, g], idxs_v.at[p], isem)

    def wait_idx(g):
        p = g % 2
        pltpu.make_async_copy(ge.at[wid, g], idxe_v.at[p], isem).wait()
        pltpu.make_async_copy(gs.at[wid, g], idxs_v.at[p], isem).wait()

    def compute_idx(g):
        p = g % 2

        def eb(i, carry):
            sl = pl.ds(i * 16, 16)
            idxe_v[p, sl] = idxe_v[p, sl] + offs_v[sl]
            return carry

        lax.fori_loop(0, CROWS // 16, eb, 0)

        def sb(i, carry):
            sl = pl.ds(i * 16, 16)
            idxs_v[p, sl] = idxs_v[p, sl] + jnp.full((16,), SP_OFF, jnp.int32)
            return carry

        lax.fori_loop(0, CPOS // 16, sb, 0)

    def fire_gathers(g):
        p, q = g % 2, g % 3
        for j in range(NSUB):
            pltpu.async_copy(table.at[idxe_v.at[p, pl.ds(j * SUB, SUB)]],
                             rows_v.at[p, pl.ds(j * SUB, SUB)], gsem)
        pltpu.async_copy(table.at[idxs_v.at[p]], acc_v.at[p], gsem)

    def wait_gathers(g):
        p = g % 2
        for j in range(NSUB):
            pltpu.make_async_copy(table.at[idxe_v.at[p, pl.ds(j * SUB, SUB)]],
                                  rows_v.at[p, pl.ds(j * SUB, SUB)], gsem).wait()
        pltpu.make_async_copy(table.at[idxs_v.at[p]], acc_v.at[p], gsem).wait()

    def fire_scatters(g):
        p, q = g % 2, g % 3
        for j in range(NSUB):
            pltpu.async_copy(rows_v.at[p, pl.ds(j * SUB, SUB)],
                             acc_sh.at[sid, q].at[sidx_v.at[j]], ssem, add=True)

    def wait_scatters(g):
        p, q = g % 2, g % 3
        for j in range(NSUB):
            pltpu.make_async_copy(rows_v.at[p, pl.ds(j * SUB, SUB)],
                                  acc_sh.at[sid, q].at[sidx_v.at[j]],
                                  ssem).wait()

    def fire_out(g):
        pltpu.async_copy(acc_sh.at[sid, g % 3],
                         acc_out.at[wid, g, :, pl.ds(0, H)], osem)

    def wait_out(g):
        pltpu.make_async_copy(acc_sh.at[sid, g % 3],
                              acc_out.at[wid, g, :, pl.ds(0, H)],
                              osem).wait()

    pltpu.sync_copy(ge.at[wid, 0], idxe_v.at[0])
    pltpu.sync_copy(gs.at[wid, 0], idxs_v.at[0])
    compute_idx(0)
    fire_gathers(0)
    fire_idx(1)

    def chunk(g, carry):
        @pl.when(g >= 1)
        def _():
            wait_scatters(g - 1)
            fire_out(g - 1)
        wait_gathers(g)
        @pl.when(g >= 2)
        def _():
            wait_out(g - 2)
        @pl.when(g <= NCHUNK - 2)
        def _():
            wait_idx(g + 1)
            compute_idx(g + 1)
            fire_gathers(g + 1)
        @pl.when(g <= NCHUNK - 3)
        def _():
            fire_idx(g + 2)
        pltpu.sync_copy(acc_v.at[g % 2], acc_sh.at[sid, g % 3])
        fire_scatters(g)
        return carry

    lax.fori_loop(0, NCHUNK, chunk, 0)
    wait_scatters(NCHUNK - 1)
    wait_out(NCHUNK - 2)
    fire_out(NCHUNK - 1)
    wait_out(NCHUNK - 1)


@functools.cache
def _sc_gather():
    return pl.kernel(
        _sc_body,
        out_type=jax.ShapeDtypeStruct((B, NCHUNK, CPOS, 4 * H), jnp.float32),
        mesh=plsc.VectorSubcoreMesh(core_axis_name="c", subcore_axis_name="s",
                                    num_cores=NC, num_subcores=NS),
        compiler_params=pltpu.CompilerParams(use_tc_tiling_on_sc=False),
        scratch_types=[
            pltpu.VMEM((2, CROWS), jnp.int32),
            pltpu.VMEM((2, CPOS), jnp.int32),
            pltpu.VMEM((NSUB, SUB), jnp.int32),
            pltpu.VMEM((CROWS,), jnp.int32),
            pltpu.VMEM((2, CROWS, H), jnp.float32),
            pltpu.VMEM((2, CPOS, H), jnp.float32),
            pltpu.VMEM_SHARED((NS, 3, CPOS, H), jnp.float32),
            pltpu.SemaphoreType.DMA,
            pltpu.SemaphoreType.DMA,
            pltpu.SemaphoreType.DMA,
            pltpu.SemaphoreType.DMA,
        ],
    )


def _sc_gather_call(table, ge, gs, sidx, offs):
    return _sc_gather()(table, ge, gs, sidx, offs)


def _final_body(acc_ref, s_ref, ab_ref, t_ref, out_ref):
    ab2 = 2.0 * ab_ref[0]
    t_col = t_ref[...]
    t_row = jnp.broadcast_to(t_col, (H, N + 1))
    ident = (lax.broadcasted_iota(jnp.int32, (H, H), 0)
             == lax.broadcasted_iota(jnp.int32, (H, H), 1)).astype(jnp.float32)
    lane65 = lax.broadcasted_iota(jnp.int32, (1, N + 1), 1)
    t0mask = jnp.where(lane65 == 0, t_row, 0.0)
    zcol = jnp.zeros((H, 1), jnp.float32)
    out_ref[0, :, 0, :] = ab2[0:1, :] + t_row
    for i in range(N):
        s = s_ref[0, pl.ds(i, 1), :]
        r = jnp.where(s <= 2, 1.0,
                      jnp.where(s == 3, 0.5,
                                jnp.where(s == 4, 1.0 / 3.0,
                                          jnp.where(s == 5, 0.25, 0.2))))
        e = acc_ref[0, i, :, pl.ds(0, H)]
        et = lax.dot_general(ident, e, (((1,), (1,)), ((), ())),
                             preferred_element_type=jnp.float32)
        et = et * r
        res = ab2[i + 1:i + 2, :] + t0mask + jnp.concatenate([zcol, et], axis=1)
        out_ref[0, :, i + 1, :] = res


def kernel(input_nodes, attn_bias, spatial_pos, input_edges, attn_edge_type,
           edge_encoder_w, edge_dis_encoder_w, spatial_pos_encoder_w, gtvd_w):
    w5 = edge_dis_encoder_w.reshape(128, H, H)[:NDIST]

    table = pl.pallas_call(
        _table_body,
        out_shape=jax.ShapeDtypeStruct((TBL_ROWS, H), jnp.float32),
    )(edge_encoder_w, w5, spatial_pos_encoder_w)

    ge = input_edges.reshape(B, NCHUNK, CROWS)
    gs = spatial_pos
    sidx = (jnp.arange(CROWS, dtype=jnp.int32) // NE).reshape(NSUB, SUB)
    offs = ((jnp.arange(CROWS, dtype=jnp.int32) // NFEAT) % NDIST) * TBL_E

    accv = _sc_gather_call(table, ge, gs, sidx, offs)

    t_col = gtvd_w.reshape(H, 1)

    out = pl.pallas_call(
        _final_body,
        grid=(B,),
        in_specs=[
            pl.BlockSpec((1, N, N, 4 * H), lambda b: (b, 0, 0, 0)),
            pl.BlockSpec((1, N, N), lambda b: (b, 0, 0)),
            pl.BlockSpec((1, N + 1, N + 1), lambda b: (b, 0, 0)),
            pl.BlockSpec((H, 1), lambda b: (0, 0)),
        ],
        out_specs=pl.BlockSpec((1, H, N + 1, N + 1), lambda b: (b, 0, 0, 0)),
        out_shape=jax.ShapeDtypeStruct((B, H, N + 1, N + 1), jnp.float32),
    )(accv, spatial_pos, attn_bias, t_col)
    return out
